# baseline (speedup 1.0000x reference)
import functools
import jax
import jax.numpy as jnp
from jax import lax
from jax.experimental import pallas as pl
from jax.experimental.pallas import tpu as pltpu
from jax.experimental.pallas import tpu_sc as plsc

VOCAB = 1000000
DIM = 64
BATCH = 16384

NC = 2
NS = 16
L = 16
NW = NC * NS
BPW = BATCH // NW
CHUNK = 128
NCHUNK = BPW // CHUNK

_mesh = plsc.VectorSubcoreMesh(
    core_axis_name="c", subcore_axis_name="s", num_cores=NC, num_subcores=NS
)


@functools.partial(
    pl.kernel,
    mesh=_mesh,
    out_type=jax.ShapeDtypeStruct((BATCH, DIM), jnp.float32),
    scratch_types=[
        pltpu.VMEM((NCHUNK, CHUNK), jnp.int32),
        pltpu.VMEM((NCHUNK, CHUNK), jnp.int32),
        pltpu.VMEM((BPW, DIM), jnp.float32),
        pltpu.VMEM((BPW, DIM), jnp.float32),
        pltpu.SemaphoreType.DMA,
        pltpu.SemaphoreType.DMA,
    ],
    compiler_params=pltpu.CompilerParams(use_tc_tiling_on_sc=False),
)
def _sc_sign_dot(mw_hbm, cw_hbm, mt_hbm, ct_hbm, out_hbm,
                 mw_v, cw_v, mrows_v, crows_v, gsem, osem):
    wid = lax.axis_index("s") * NC + lax.axis_index("c")
    base = wid * NCHUNK

    pltpu.sync_copy(mw_hbm.at[pl.ds(base, NCHUNK)], mw_v)
    pltpu.sync_copy(cw_hbm.at[pl.ds(base, NCHUNK)], cw_v)

    copies = []
    for j in range(NCHUNK):
        copies.append(pltpu.async_copy(
            mt_hbm.at[mw_v.at[j]], mrows_v.at[pl.ds(j * CHUNK, CHUNK)], gsem))
        copies.append(pltpu.async_copy(
            ct_hbm.at[cw_v.at[j]], crows_v.at[pl.ds(j * CHUNK, CHUNK)], gsem))
    for cp in copies:
        cp.wait()

    def body(r, carry):
        for c in range(DIM // L):
            a = mrows_v[r, pl.ds(c * L, L)]
            b = crows_v[r, pl.ds(c * L, L)]
            p = a * b
            mrows_v[r, pl.ds(c * L, L)] = p / jnp.abs(p)
        return carry
    lax.fori_loop(0, BPW, body, 0)

    pltpu.async_copy(mrows_v, out_hbm.at[pl.ds(wid * BPW, BPW)], osem).wait()


def kernel(main_words, ctx_words, main_table, ctx_table):
    mw = main_words.astype(jnp.int32).reshape(BATCH // CHUNK, CHUNK)
    cw = ctx_words.astype(jnp.int32).reshape(BATCH // CHUNK, CHUNK)
    return _sc_sign_dot(mw, cw, main_table, ctx_table)


# speedup vs baseline: 1.5839x; 1.5839x over previous
import functools
import jax
import jax.numpy as jnp
from jax import lax
from jax.experimental import pallas as pl
from jax.experimental.pallas import tpu as pltpu
from jax.experimental.pallas import tpu_sc as plsc

VOCAB = 1000000
DIM = 64
BATCH = 16384

NC = 2
NS = 16
L = 16
NW = NC * NS
BPW = BATCH // NW
CR = 256
NPASS = BPW // CR

_mesh = plsc.VectorSubcoreMesh(
    core_axis_name="c", subcore_axis_name="s", num_cores=NC, num_subcores=NS
)


@functools.partial(
    pl.kernel,
    mesh=_mesh,
    out_type=jax.ShapeDtypeStruct((BATCH, DIM), jnp.float32),
    scratch_types=[
        pltpu.VMEM((BPW,), jnp.int32),
        pltpu.VMEM((BPW,), jnp.int32),
        pltpu.VMEM((CR, DIM), jnp.float32),
        pltpu.VMEM((CR, DIM), jnp.float32),
        pltpu.SemaphoreType.DMA,
        pltpu.SemaphoreType.DMA,
        pltpu.SemaphoreType.DMA,
    ],
)
def _sc_sign_dot(mw_hbm, cw_hbm, mt_hbm, ct_hbm, out_hbm,
                 mw_v, cw_v, mrows_v, crows_v, msem, csem, osem):
    wid = lax.axis_index("s") * NC + lax.axis_index("c")
    base = wid * BPW

    pltpu.sync_copy(mw_hbm.at[pl.ds(base, BPW)], mw_v)
    pltpu.sync_copy(cw_hbm.at[pl.ds(base, BPW)], cw_v)

    for p in range(NPASS):
        def issue(g, carry):
            r0 = g * L
            mv = mw_v[pl.ds(p * CR + r0, L)]
            cv = cw_v[pl.ds(p * CR + r0, L)]
            for lane in range(L):
                pltpu.async_copy(
                    mt_hbm.at[pl.ds(mv[lane], 1)],
                    mrows_v.at[pl.ds(r0 + lane, 1)], msem)
                pltpu.async_copy(
                    ct_hbm.at[pl.ds(cv[lane], 1)],
                    crows_v.at[pl.ds(r0 + lane, 1)], csem)
            return carry
        lax.fori_loop(0, CR // L, issue, 0)

        pltpu.make_async_copy(mt_hbm.at[pl.ds(0, CR)], mrows_v, msem).wait()
        pltpu.make_async_copy(ct_hbm.at[pl.ds(0, CR)], crows_v, csem).wait()

        def body(r, carry):
            for c in range(DIM // L):
                a = mrows_v[r, pl.ds(c * L, L)]
                b = crows_v[r, pl.ds(c * L, L)]
                prod = a * b
                mrows_v[r, pl.ds(c * L, L)] = prod / jnp.abs(prod)
            return carry
        lax.fori_loop(0, CR, body, 0)

        pltpu.async_copy(
            mrows_v, out_hbm.at[pl.ds(base + p * CR, CR)], osem).wait()


def kernel(main_words, ctx_words, main_table, ctx_table):
    return _sc_sign_dot(main_words.astype(jnp.int32), ctx_words.astype(jnp.int32),
                        main_table, ctx_table)
